# baseline (speedup 1.0000x reference)
import functools

import jax
import jax.numpy as jnp
from jax import lax
from jax.experimental import pallas as pl
from jax.experimental.pallas import tpu as pltpu
from jax.experimental.pallas import tpu_sc as plsc

_info = plsc.get_sparse_core_info()
_NC, _NS = _info.num_cores, _info.num_subcores
_NW = _NC * _NS
_L = 16

_N = 4 * 8192
_D = 1024
_NG = _D // _L
_TPW = _N // _NW
_K = 16
_NCHUNK = _TPW // _K


def _sc_body(ids_hbm, table_hbm, out_hbm, idx_v, tbl_v, buf0, buf1, ws0,
             ws1):
    wid = lax.axis_index("s") * _NC + lax.axis_index("c")
    base = wid * _TPW

    pltpu.sync_copy(table_hbm, tbl_v)
    pltpu.sync_copy(ids_hbm.at[pl.ds(base, _TPW)], idx_v)

    bufs = (buf0, buf1)
    wsems = (ws0, ws1)

    def materialize(c, b):
        buf = bufs[b]
        scales = []
        for g in range(_K // _L):
            ids_vec = idx_v[pl.ds(c * _K + g * _L, _L)]
            for t in range(_L):
                s_i = jnp.full((_L,), ids_vec[t], jnp.int32)
                scales.append(s_i.astype(jnp.float32))

        def col_group(j, carry):
            w0 = tbl_v[0, pl.ds(j * _L, _L)]
            d = tbl_v[1, pl.ds(j * _L, _L)] - w0
            for k in range(_K):
                buf[k, pl.ds(j * _L, _L)] = w0 + scales[k] * d
            return carry

        lax.fori_loop(0, _NG, col_group, 0)

    def start_write(c, b):
        pltpu.async_copy(
            bufs[b], out_hbm.at[pl.ds(base + c * _K, _K)], wsems[b])

    def wait_write(c, b):
        pltpu.make_async_copy(
            bufs[b], out_hbm.at[pl.ds(base + c * _K, _K)], wsems[b]).wait()

    materialize(0, 0)
    start_write(0, 0)
    materialize(1, 1)
    start_write(1, 1)

    def body(p, carry):
        c0 = 2 * p + 2
        wait_write(c0 - 2, 0)
        materialize(c0, 0)
        start_write(c0, 0)
        wait_write(c0 - 1, 1)
        materialize(c0 + 1, 1)
        start_write(c0 + 1, 1)
        return carry

    lax.fori_loop(0, _NCHUNK // 2 - 1, body, 0)

    wait_write(_NCHUNK - 2, 0)
    wait_write(_NCHUNK - 1, 1)


@jax.jit
def _lookup(ids_flat, table):
    mesh = plsc.VectorSubcoreMesh(core_axis_name="c", subcore_axis_name="s")
    f = functools.partial(
        pl.kernel,
        out_type=jax.ShapeDtypeStruct((_N, _D), jnp.float32),
        mesh=mesh,
        scratch_types=[
            pltpu.VMEM((_TPW,), jnp.int32),
            pltpu.VMEM((2, _D), jnp.float32),
            pltpu.VMEM((_K, _D), jnp.float32),
            pltpu.VMEM((_K, _D), jnp.float32),
            pltpu.SemaphoreType.DMA,
            pltpu.SemaphoreType.DMA,
        ],
    )(_sc_body)
    return f(ids_flat, table)


def kernel(token_type_ids, embedding_weight):
    ids_flat = token_type_ids.astype(jnp.int32).reshape(_N)
    out = _lookup(ids_flat, embedding_weight)
    return out.reshape(token_type_ids.shape + (_D,))


# speedup vs baseline: 1.1674x; 1.1674x over previous
import functools

import jax
import jax.numpy as jnp
from jax import lax
from jax.experimental import pallas as pl
from jax.experimental.pallas import tpu as pltpu
from jax.experimental.pallas import tpu_sc as plsc

_info = plsc.get_sparse_core_info()
_NC, _NS = _info.num_cores, _info.num_subcores
_NW = _NC * _NS
_L = 16

_N = 4 * 8192
_D = 1024
_NG = _D // _L
_TPW = _N // _NW
_K = 32
_NCHUNK = _TPW // _K


def _sc_body(ids_hbm, table_hbm, out_hbm, idx_v, tbl_v, buf0, buf1, ws0,
             ws1):
    wid = lax.axis_index("s") * _NC + lax.axis_index("c")
    base = wid * _TPW

    h_t = pltpu.async_copy(table_hbm, tbl_v, ws0)
    h_i = pltpu.async_copy(ids_hbm.at[pl.ds(base, _TPW)], idx_v, ws1)
    h_t.wait()
    h_i.wait()

    bufs = (buf0, buf1)
    wsems = (ws0, ws1)

    def materialize(c, b, r0=0, nr=_K):
        buf = bufs[b]
        scales = []
        for g in range(nr // _L):
            ids_vec = idx_v[pl.ds(c * _K + r0 + g * _L, _L)]
            for t in range(_L):
                s_i = jnp.full((_L,), ids_vec[t], jnp.int32)
                scales.append(s_i.astype(jnp.float32))

        def col_group(j, carry):
            w0 = tbl_v[0, pl.ds(j * _L, _L)]
            d = tbl_v[1, pl.ds(j * _L, _L)] - w0
            for k in range(nr):
                buf[r0 + k, pl.ds(j * _L, _L)] = w0 + scales[k] * d
            return carry

        lax.fori_loop(0, _NG, col_group, 0)

    def start_write(c, b, r0=0, nr=_K):
        pltpu.async_copy(
            bufs[b].at[pl.ds(r0, nr)],
            out_hbm.at[pl.ds(base + c * _K + r0, nr)], wsems[b])

    def wait_write_full(b):
        pltpu.make_async_copy(
            bufs[b], out_hbm.at[pl.ds(base, _K)], wsems[b]).wait()

    _H = _K // 2
    materialize(0, 0, 0, _H)
    start_write(0, 0, 0, _H)
    materialize(0, 0, _H, _H)
    start_write(0, 0, _H, _H)
    materialize(1, 1)
    start_write(1, 1)

    def body(p, carry):
        c0 = 2 * p + 2
        wait_write_full(0)
        materialize(c0, 0)
        start_write(c0, 0)
        wait_write_full(1)
        materialize(c0 + 1, 1)
        start_write(c0 + 1, 1)
        return carry

    lax.fori_loop(0, _NCHUNK // 2 - 2, body, 0)

    wait_write_full(0)
    materialize(_NCHUNK - 2, 0)
    start_write(_NCHUNK - 2, 0)
    wait_write_full(1)
    materialize(_NCHUNK - 1, 1, 0, _H)
    start_write(_NCHUNK - 1, 1, 0, _H)
    materialize(_NCHUNK - 1, 1, _H, _H)
    start_write(_NCHUNK - 1, 1, _H, _H)
    wait_write_full(0)
    wait_write_full(1)


@jax.jit
def _lookup(ids_flat, table):
    mesh = plsc.VectorSubcoreMesh(core_axis_name="c", subcore_axis_name="s")
    f = functools.partial(
        pl.kernel,
        out_type=jax.ShapeDtypeStruct((_N, _D), jnp.float32),
        mesh=mesh,
        scratch_types=[
            pltpu.VMEM((_TPW,), jnp.int32),
            pltpu.VMEM((2, _D), jnp.float32),
            pltpu.VMEM((_K, _D), jnp.float32),
            pltpu.VMEM((_K, _D), jnp.float32),
            pltpu.SemaphoreType.DMA,
            pltpu.SemaphoreType.DMA,
        ],
    )(_sc_body)
    return f(ids_flat, table)


def kernel(token_type_ids, embedding_weight):
    ids_flat = token_type_ids.astype(jnp.int32).reshape(_N)
    out = _lookup(ids_flat, embedding_weight)
    return out.reshape(token_type_ids.shape + (_D,))


# speedup vs baseline: 1.1858x; 1.0158x over previous
import functools

import jax
import jax.numpy as jnp
from jax import lax
from jax.experimental import pallas as pl
from jax.experimental.pallas import tpu as pltpu
from jax.experimental.pallas import tpu_sc as plsc

_info = plsc.get_sparse_core_info()
_NC, _NS = _info.num_cores, _info.num_subcores
_NW = _NC * _NS
_L = 16

_N = 4 * 8192
_D = 1024
_NG = _D // _L
_TPW = _N // _NW
_K = 32
_NCHUNK = _TPW // _K


def _sc_body(ids_hbm, table_hbm, out_hbm, idx_v, tbl_v, buf0, buf1, ws0,
             ws1):
    wid = lax.axis_index("s") * _NC + lax.axis_index("c")
    base = wid * _TPW

    h_t = pltpu.async_copy(table_hbm, tbl_v, ws0)
    h_i = pltpu.async_copy(ids_hbm.at[pl.ds(base, _TPW)], idx_v, ws1)
    h_t.wait()
    h_i.wait()

    bufs = (buf0, buf1)
    wsems = (ws0, ws1)

    def materialize(c, b, r0=0, nr=_K):
        buf = bufs[b]
        scales = []
        for g in range(nr // _L):
            ids_vec = idx_v[pl.ds(c * _K + r0 + g * _L, _L)]
            for t in range(_L):
                s_i = jnp.full((_L,), ids_vec[t], jnp.int32)
                scales.append(s_i.astype(jnp.float32))

        def col_group(j, carry):
            w0 = tbl_v[0, pl.ds(j * _L, _L)]
            d = tbl_v[1, pl.ds(j * _L, _L)] - w0
            for k in range(nr):
                buf[r0 + k, pl.ds(j * _L, _L)] = w0 + scales[k] * d
            return carry

        lax.fori_loop(0, _NG, col_group, 0)

    def start_write(c, b):
        pltpu.async_copy(
            bufs[b], out_hbm.at[pl.ds(base + c * _K, _K)], wsems[b])

    def wait_write(c, b):
        pltpu.make_async_copy(
            bufs[b], out_hbm.at[pl.ds(base + c * _K, _K)], wsems[b]).wait()

    materialize(0, 0)
    start_write(0, 0)
    materialize(1, 1)
    start_write(1, 1)

    def body(p, carry):
        c0 = 2 * p + 2
        wait_write(c0 - 2, 0)
        materialize(c0, 0)
        start_write(c0, 0)
        wait_write(c0 - 1, 1)
        materialize(c0 + 1, 1)
        start_write(c0 + 1, 1)
        return carry

    lax.fori_loop(0, _NCHUNK // 2 - 1, body, 0)

    wait_write(_NCHUNK - 2, 0)
    wait_write(_NCHUNK - 1, 1)


@jax.jit
def _lookup(ids_flat, table):
    mesh = plsc.VectorSubcoreMesh(core_axis_name="c", subcore_axis_name="s")
    f = functools.partial(
        pl.kernel,
        out_type=jax.ShapeDtypeStruct((_N, _D), jnp.float32),
        mesh=mesh,
        scratch_types=[
            pltpu.VMEM((_TPW,), jnp.int32),
            pltpu.VMEM((2, _D), jnp.float32),
            pltpu.VMEM((_K, _D), jnp.float32),
            pltpu.VMEM((_K, _D), jnp.float32),
            pltpu.SemaphoreType.DMA,
            pltpu.SemaphoreType.DMA,
        ],
    )(_sc_body)
    return f(ids_flat, table)


def kernel(token_type_ids, embedding_weight):
    ids_flat = token_type_ids.astype(jnp.int32).reshape(_N)
    out = _lookup(ids_flat, embedding_weight)
    return out.reshape(token_type_ids.shape + (_D,))


# speedup vs baseline: 1.2217x; 1.0303x over previous
import functools

import jax
import jax.numpy as jnp
from jax import lax
from jax.experimental import pallas as pl
from jax.experimental.pallas import tpu as pltpu
from jax.experimental.pallas import tpu_sc as plsc

_info = plsc.get_sparse_core_info()
_NC, _NS = _info.num_cores, _info.num_subcores
_NW = _NC * _NS
_L = 16

_N = 4 * 8192
_D = 1024
_NG = _D // _L
_TPW = _N // _NW
_K = 48
_KT = 16
_NFULL = (_TPW - _KT) // _K


def _sc_body(ids_hbm, table_hbm, out_hbm, idx_v, tbl_v, buf0, buf1, ws0,
             ws1):
    wid = lax.axis_index("s") * _NC + lax.axis_index("c")
    base = wid * _TPW

    h_t = pltpu.async_copy(table_hbm, tbl_v, ws0)
    h_i = pltpu.async_copy(ids_hbm.at[pl.ds(base, _TPW)], idx_v, ws1)
    h_t.wait()
    h_i.wait()

    bufs = (buf0, buf1)
    wsems = (ws0, ws1)

    def materialize(c, b, nr=_K):
        buf = bufs[b]
        scales = []
        for g in range(nr // _L):
            ids_vec = idx_v[pl.ds(c * _K + g * _L, _L)]
            for t in range(_L):
                s_i = jnp.full((_L,), ids_vec[t], jnp.int32)
                scales.append(s_i.astype(jnp.float32))

        def col_group(j, carry):
            w0 = tbl_v[0, pl.ds(j * _L, _L)]
            d = tbl_v[1, pl.ds(j * _L, _L)] - w0
            for k in range(nr):
                buf[k, pl.ds(j * _L, _L)] = w0 + scales[k] * d
            return carry

        lax.fori_loop(0, _NG, col_group, 0)

    def start_write(c, b, nr=_K):
        pltpu.async_copy(
            bufs[b].at[pl.ds(0, nr)],
            out_hbm.at[pl.ds(base + c * _K, nr)], wsems[b])

    def wait_write(c, b, nr=_K):
        pltpu.make_async_copy(
            bufs[b].at[pl.ds(0, nr)],
            out_hbm.at[pl.ds(base + c * _K, nr)], wsems[b]).wait()

    materialize(0, 0)
    start_write(0, 0)
    materialize(1, 1)
    start_write(1, 1)

    def body(p, carry):
        c0 = 2 * p + 2
        wait_write(c0 - 2, 0)
        materialize(c0, 0)
        start_write(c0, 0)
        wait_write(c0 - 1, 1)
        materialize(c0 + 1, 1)
        start_write(c0 + 1, 1)
        return carry

    lax.fori_loop(0, (_NFULL - 1) // 2 - 1, body, 0)

    wait_write(_NFULL - 3, 0)
    materialize(_NFULL - 1, 0)
    start_write(_NFULL - 1, 0)
    wait_write(_NFULL - 2, 1)
    materialize(_NFULL, 1, _KT)
    start_write(_NFULL, 1, _KT)
    wait_write(_NFULL - 1, 0)
    wait_write(_NFULL, 1, _KT)


@jax.jit
def _lookup(ids_flat, table):
    mesh = plsc.VectorSubcoreMesh(core_axis_name="c", subcore_axis_name="s")
    f = functools.partial(
        pl.kernel,
        out_type=jax.ShapeDtypeStruct((_N, _D), jnp.float32),
        mesh=mesh,
        scratch_types=[
            pltpu.VMEM((_TPW,), jnp.int32),
            pltpu.VMEM((2, _D), jnp.float32),
            pltpu.VMEM((_K, _D), jnp.float32),
            pltpu.VMEM((_K, _D), jnp.float32),
            pltpu.SemaphoreType.DMA,
            pltpu.SemaphoreType.DMA,
        ],
    )(_sc_body)
    return f(ids_flat, table)


def kernel(token_type_ids, embedding_weight):
    ids_flat = token_type_ids.astype(jnp.int32).reshape(_N)
    out = _lookup(ids_flat, embedding_weight)
    return out.reshape(token_type_ids.shape + (_D,))


# speedup vs baseline: 1.2224x; 1.0006x over previous
import functools

import jax
import jax.numpy as jnp
from jax import lax
from jax.experimental import pallas as pl
from jax.experimental.pallas import tpu as pltpu
from jax.experimental.pallas import tpu_sc as plsc

_info = plsc.get_sparse_core_info()
_NC, _NS = _info.num_cores, _info.num_subcores
_NW = _NC * _NS
_L = 16

_N = 4 * 8192
_D = 1024
_NG = _D // _L
_TPW = _N // _NW
_K = 56
_KT = 16
_NFULL = (_TPW - _KT) // _K
_IDXPAD = _NFULL * _K + _L


def _sc_body(ids_hbm, table_hbm, out_hbm, idx_v, tbl_v, buf0, buf1, ws0,
             ws1):
    wid = lax.axis_index("s") * _NC + lax.axis_index("c")
    base = wid * _TPW

    h_t = pltpu.async_copy(table_hbm, tbl_v, ws0)
    h_i = pltpu.async_copy(
        ids_hbm.at[pl.ds(base, _TPW)], idx_v.at[pl.ds(0, _TPW)], ws1)
    h_t.wait()
    h_i.wait()

    bufs = (buf0, buf1)
    wsems = (ws0, ws1)

    def materialize(c, b, nr=_K):
        buf = bufs[b]
        scales = []
        for g in range((nr + _L - 1) // _L):
            ids_vec = idx_v[pl.ds(c * _K + g * _L, _L)]
            for t in range(min(_L, nr - g * _L)):
                s_i = jnp.full((_L,), ids_vec[t], jnp.int32)
                scales.append(s_i.astype(jnp.float32))

        def col_group(j, carry):
            w0 = tbl_v[0, pl.ds(j * _L, _L)]
            d = tbl_v[1, pl.ds(j * _L, _L)] - w0
            for k in range(nr):
                buf[k, pl.ds(j * _L, _L)] = w0 + scales[k] * d
            return carry

        lax.fori_loop(0, _NG, col_group, 0)

    def start_write(c, b, nr=_K):
        pltpu.async_copy(
            bufs[b].at[pl.ds(0, nr)],
            out_hbm.at[pl.ds(base + c * _K, nr)], wsems[b])

    def wait_write(c, b, nr=_K):
        pltpu.make_async_copy(
            bufs[b].at[pl.ds(0, nr)],
            out_hbm.at[pl.ds(base + c * _K, nr)], wsems[b]).wait()

    materialize(0, 0)
    start_write(0, 0)
    materialize(1, 1)
    start_write(1, 1)

    def body(p, carry):
        c0 = 2 * p + 2
        wait_write(c0 - 2, 0)
        materialize(c0, 0)
        start_write(c0, 0)
        wait_write(c0 - 1, 1)
        materialize(c0 + 1, 1)
        start_write(c0 + 1, 1)
        return carry

    lax.fori_loop(0, (_NFULL - 4) // 2, body, 0)

    wait_write(_NFULL - 4, 0)
    materialize(_NFULL - 2, 0)
    start_write(_NFULL - 2, 0)
    wait_write(_NFULL - 3, 1)
    materialize(_NFULL - 1, 1)
    start_write(_NFULL - 1, 1)
    wait_write(_NFULL - 2, 0)
    materialize(_NFULL, 0, _KT)
    start_write(_NFULL, 0, _KT)
    wait_write(_NFULL - 1, 1)
    wait_write(_NFULL, 0, _KT)


@jax.jit
def _lookup(ids_flat, table):
    mesh = plsc.VectorSubcoreMesh(core_axis_name="c", subcore_axis_name="s")
    f = functools.partial(
        pl.kernel,
        out_type=jax.ShapeDtypeStruct((_N, _D), jnp.float32),
        mesh=mesh,
        scratch_types=[
            pltpu.VMEM((_IDXPAD,), jnp.int32),
            pltpu.VMEM((2, _D), jnp.float32),
            pltpu.VMEM((_K, _D), jnp.float32),
            pltpu.VMEM((_K, _D), jnp.float32),
            pltpu.SemaphoreType.DMA,
            pltpu.SemaphoreType.DMA,
        ],
    )(_sc_body)
    return f(ids_flat, table)


def kernel(token_type_ids, embedding_weight):
    ids_flat = token_type_ids.astype(jnp.int32).reshape(_N)
    out = _lookup(ids_flat, embedding_weight)
    return out.reshape(token_type_ids.shape + (_D,))
